# baseline (speedup 1.0000x reference)
import functools

import jax
import jax.numpy as jnp
from jax import lax
from jax.experimental import pallas as pl
from jax.experimental.pallas import tpu as pltpu
from jax.experimental.pallas import tpu_sc as plsc

_ROWS = 8192
_DIM = 1024
_N_WORKERS = 32
_ROWS_PER_WORKER = _ROWS // _N_WORKERS
_CHUNK_ROWS = 32
_N_CHUNKS = _ROWS_PER_WORKER // _CHUNK_ROWS

_mesh = plsc.VectorSubcoreMesh(core_axis_name="c", subcore_axis_name="s")


@functools.partial(
    pl.kernel,
    mesh=_mesh,
    out_type=jax.ShapeDtypeStruct((_ROWS, _DIM), jnp.float32),
    scratch_types=[
        pltpu.VMEM((_CHUNK_ROWS, _DIM), jnp.float32),
        pltpu.VMEM((_CHUNK_ROWS, _DIM), jnp.float32),
        pltpu.SemaphoreType.DMA,
        pltpu.SemaphoreType.DMA,
        pltpu.SemaphoreType.DMA,
        pltpu.SemaphoreType.DMA,
    ],
)
def _sc_copy(w_hbm, out_hbm, buf0, buf1, in_sem0, in_sem1, out_sem0, out_sem1):
    wid = lax.axis_index("s") * 2 + lax.axis_index("c")
    base = wid * _ROWS_PER_WORKER
    bufs = (buf0, buf1)
    in_sems = (in_sem0, in_sem1)
    out_sems = (out_sem0, out_sem1)

    in_copies = [
        pltpu.make_async_copy(
            w_hbm.at[pl.ds(base + k * _CHUNK_ROWS, _CHUNK_ROWS)],
            bufs[k % 2],
            in_sems[k % 2],
        )
        for k in range(_N_CHUNKS)
    ]
    out_copies = [
        pltpu.make_async_copy(
            bufs[k % 2],
            out_hbm.at[pl.ds(base + k * _CHUNK_ROWS, _CHUNK_ROWS)],
            out_sems[k % 2],
        )
        for k in range(_N_CHUNKS)
    ]

    in_copies[0].start()
    for k in range(_N_CHUNKS):
        if k + 1 < _N_CHUNKS:
            if k >= 1:
                out_copies[k - 1].wait()
            in_copies[k + 1].start()
        in_copies[k].wait()
        out_copies[k].start()
    out_copies[_N_CHUNKS - 2].wait()
    out_copies[_N_CHUNKS - 1].wait()


def kernel(x, W):
    out = _sc_copy(W)
    return out[None, :, :]


# speedup vs baseline: 2.0361x; 2.0361x over previous
import jax
import jax.numpy as jnp
from jax.experimental import pallas as pl
from jax.experimental.pallas import tpu as pltpu

_N_CHUNKS = 4
_CHUNK_ROWS = 2048


def _dma_pipeline_kernel(w_ref, o_ref, b0, b1, b2, b3, in_sem, out_sem):
    bufs = (b0, b1, b2, b3)
    in_copies = [
        pltpu.make_async_copy(
            w_ref.at[pl.ds(k * _CHUNK_ROWS, _CHUNK_ROWS)], bufs[k], in_sem
        )
        for k in range(_N_CHUNKS)
    ]
    out_copies = [
        pltpu.make_async_copy(
            bufs[k], o_ref.at[pl.ds(k * _CHUNK_ROWS, _CHUNK_ROWS)], out_sem
        )
        for k in range(_N_CHUNKS)
    ]
    for c in in_copies:
        c.start()
    for k in range(_N_CHUNKS):
        in_copies[k].wait()
        out_copies[k].start()
    for c in out_copies:
        c.wait()


def kernel(x, W):
    seq_len = x.shape[1]
    rows, dim = W.shape
    out = pl.pallas_call(
        _dma_pipeline_kernel,
        in_specs=[pl.BlockSpec(memory_space=pl.ANY)],
        out_specs=pl.BlockSpec(memory_space=pl.ANY),
        out_shape=jax.ShapeDtypeStruct((seq_len, dim), W.dtype),
        scratch_shapes=[pltpu.VMEM((_CHUNK_ROWS, dim), W.dtype)] * _N_CHUNKS
        + [pltpu.SemaphoreType.DMA, pltpu.SemaphoreType.DMA],
        compiler_params=pltpu.CompilerParams(vmem_limit_bytes=64 * 1024 * 1024),
    )(W)
    return out[None, :, :]
